# baseline (speedup 1.0000x reference)
import functools

import jax
import jax.numpy as jnp
import numpy as np
from jax.experimental import pallas as pl
from jax.experimental.pallas import tpu as pltpu

B, T, N, D, M, K, H = 16, 48, 256, 64, 4096, 8, 4
DH = D // H
ND = N * D
TAU_TIME = 2.0
TPAD = 64
CONV_K = 12
O_BLK = 2048
M_BLK = 256
BK = B * K


_ERFC_P = [2.326819970068386e-2, -1.387039388740657e-1, 3.687424674597105e-1,
           -5.824733027278666e-1, 6.210004621745983e-1, -4.944515323274145e-1,
           3.404879937665872e-1, -2.741127028184656e-1, 5.638259427386472e-1]
_ERFC_R = [-1.047766399936249e+1, 1.297719955372516e+1, -7.495518717768503e+0,
           2.921019019210786e+0, -1.015265279202700e+0, 4.218463358204948e-1,
           -2.820767439740514e-1, 5.641895067754075e-1]


def _horner(y, coeffs):
    p = jnp.full_like(y, coeffs[0])
    for c in coeffs[1:]:
        p = p * y + c
    return p


def _erfc(a):
    ax = jnp.abs(a)
    z = jnp.exp(-a * a)
    q = 1.0 / ax
    y2 = q * q
    p = jnp.where(ax < 2.0, _horner(y2, _ERFC_P), _horner(y2, _ERFC_R))
    y = z * q * p
    y = jnp.where(a * a > 88.72283905206835, 0.0, y)
    big = jnp.where(a < 0.0, 2.0 - y, y)
    small = 1.0 - jax.lax.erf(a)
    return jnp.where(ax > 1.0, big, small)


_SQRT_HALF = float(np.float32(0.7071067811865476))


def _gelu(x):
    return 0.5 * x * _erfc(-x * _SQRT_HALF)


def _encoder_body(xp_ref, wj_ref, dwb_ref, pw_ref, pwb_ref, lng_ref, lnb_ref,
                  q_ref, xg_ref):
    step = pl.program_id(0)

    @pl.when(step == 0)
    def _():
        acc = jnp.zeros((B * TPAD, N), jnp.float32)
        for j in range(CONV_K):
            acc = acc + xp_ref[j:j + B * TPAD, :] * wj_ref[j:j + 1, :]
        acc = acc + dwb_ref[...]
        xg_ref[...] = _gelu(acc)

    xg = xg_ref[...]
    y = jax.lax.dot_general(xg, pw_ref[...], (((1,), (1,)), ((), ())),
                            preferred_element_type=jnp.float32)
    y = _gelu(y + pwb_ref[...])
    row = jax.lax.broadcasted_iota(jnp.int32, (B * TPAD, 1), 0)
    valid = (row % TPAD) < (T + 1)
    y = jnp.where(valid, y, 0.0)
    qraw = jnp.sum(y.reshape(B, TPAD, O_BLK), axis=1) / jnp.float32(T + 1)
    qn = qraw.reshape(B, O_BLK // D, D)
    mu = jnp.mean(qn, axis=-1, keepdims=True)
    var = jnp.mean((qn - mu) * (qn - mu), axis=-1, keepdims=True)
    qn = (qn - mu) / jnp.sqrt(var + 1e-5)
    qn = qn * lng_ref[...].reshape(1, 1, D) + lnb_ref[...].reshape(1, 1, D)
    q_ref[...] = qn.reshape(B, O_BLK)


def _run_encoder(x_scalar, dw_w, dw_b, pw_w, pw_b, ln_g, ln_b):
    xp = jnp.pad(x_scalar, ((0, 0), (6, TPAD - 6 - T), (0, 0)))
    xp = xp.reshape(B * TPAD, N)
    xp = jnp.pad(xp, ((0, 16), (0, 0)))
    wj = jnp.pad(jnp.transpose(dw_w[:, 0, :], (1, 0)), ((0, 4), (0, 0)))
    grid = ND // O_BLK
    return pl.pallas_call(
        _encoder_body,
        grid=(grid,),
        in_specs=[
            pl.BlockSpec((B * TPAD + 16, N), lambda o: (0, 0)),
            pl.BlockSpec((16, N), lambda o: (0, 0)),
            pl.BlockSpec((1, N), lambda o: (0, 0)),
            pl.BlockSpec((O_BLK, N), lambda o: (o, 0)),
            pl.BlockSpec((1, O_BLK), lambda o: (0, o)),
            pl.BlockSpec((1, D), lambda o: (0, 0)),
            pl.BlockSpec((1, D), lambda o: (0, 0)),
        ],
        out_specs=pl.BlockSpec((B, O_BLK), lambda o: (0, o)),
        out_shape=jax.ShapeDtypeStruct((B, ND), jnp.float32),
        scratch_shapes=[pltpu.VMEM((B * TPAD, N), jnp.float32)],
    )(xp, wj, dw_b.reshape(1, N), pw_w, pw_b.reshape(1, ND),
      ln_g.reshape(1, D), ln_b.reshape(1, D))


def _sim_body(q_ref, mb_ref, msea_ref, myr_ref, seaq_ref, yrq_ref,
              idx_ref, qf_ref, sim_ref):
    step = pl.program_id(0)
    nsteps = pl.num_programs(0)

    @pl.when(step == 0)
    def _():
        q = q_ref[...]
        nrm = jnp.sqrt(jnp.sum(q * q, axis=1, keepdims=True))
        qf_ref[...] = q / jnp.maximum(nrm, 1e-12)

    mb = mb_ref[...]
    nrm = jnp.sqrt(jnp.sum(mb * mb, axis=1, keepdims=True))
    mf = mb / jnp.maximum(nrm, 1e-12)
    cos = jax.lax.dot_general(qf_ref[...], mf, (((1,), (1,)), ((), ())),
                              preferred_element_type=jnp.float32)
    smask = (seaq_ref[...] == msea_ref[...]).astype(jnp.float32)
    sim = cos * smask + (1.0 - smask) * (-10000.0)
    dy = jnp.abs(yrq_ref[...] - myr_ref[...])
    sim = sim * (1.0 - 0.5 * jnp.exp(-dy / TAU_TIME))
    sim_ref[:, pl.ds(step * M_BLK, M_BLK)] = sim

    @pl.when(step == nsteps - 1)
    def _():
        sims = sim_ref[...]
        iota = jax.lax.broadcasted_iota(jnp.int32, (B, M), 1)
        for k in range(K):
            v = jnp.max(sims, axis=1, keepdims=True)
            cand = jnp.where(sims == v, iota, M)
            sel = jnp.min(cand, axis=1, keepdims=True)
            idx_ref[:, k:k + 1] = sel
            sims = jnp.where(iota == sel, -jnp.inf, sims)


def _run_sim_topk(q, mb, memory_seasons, memory_years, season_q, year_q):
    grid = M // M_BLK
    return pl.pallas_call(
        _sim_body,
        grid=(grid,),
        in_specs=[
            pl.BlockSpec((B, ND), lambda i: (0, 0)),
            pl.BlockSpec((M_BLK, ND), lambda i: (i, 0)),
            pl.BlockSpec((1, M_BLK), lambda i: (0, i)),
            pl.BlockSpec((1, M_BLK), lambda i: (0, i)),
            pl.BlockSpec((B, 1), lambda i: (0, 0)),
            pl.BlockSpec((B, 1), lambda i: (0, 0)),
        ],
        out_specs=pl.BlockSpec((B, 128), lambda i: (0, 0)),
        out_shape=jax.ShapeDtypeStruct((B, 128), jnp.int32),
        scratch_shapes=[pltpu.VMEM((B, ND), jnp.float32),
                        pltpu.VMEM((B, M), jnp.float32)],
    )(q, mb, memory_seasons.reshape(1, M), memory_years.reshape(1, M),
      season_q.reshape(B, 1), year_q.reshape(B, 1))


GATHER_SPLIT = 64
GATHER_W = 128
_SUB = ND // GATHER_SPLIT


def _run_gather(mb, idx_flat):
    from jax.experimental.pallas import tpu_sc as plsc

    mesh = plsc.VectorSubcoreMesh(core_axis_name="core",
                                  subcore_axis_name="subcore")
    nsub = BK * GATHER_SPLIT
    idx8 = (idx_flat[:, None] * GATHER_SPLIT
            + jnp.arange(GATHER_SPLIT, dtype=jnp.int32)[None, :]
            ).reshape(1, nsub)
    mb8 = mb.reshape(M * GATHER_SPLIT, _SUB)

    @functools.partial(
        pl.kernel,
        out_type=jax.ShapeDtypeStruct((nsub, _SUB), jnp.float32),
        mesh=mesh,
    )
    def gk(x_hbm, i_hbm, o_hbm):
        def body(i_vmem, o_vmem):
            pltpu.sync_copy(x_hbm.at[i_vmem.at[0]], o_vmem)

        pltpu.emit_pipeline(
            body,
            grid=(nsub // GATHER_W,),
            in_specs=[pl.BlockSpec((1, GATHER_W), lambda i: (0, i))],
            out_specs=[pl.BlockSpec((GATHER_W, _SUB), lambda i: (i, 0))],
            core_axis_name="subcore",
            dimension_semantics=(pltpu.PARALLEL,),
        )(i_hbm, o_hbm)

    return gk(mb8, idx8).reshape(BK, ND)


ATT_C = 8


def _attn_body(q_ref, r_ref, wq_ref, bq_ref, wk_ref, bk_ref, wv_ref, bv_ref,
               wo_ref, bo_ref, pw_ref, pb_ref, out_ref):
    BN = (B * N) // ATT_C
    c = pl.program_id(0)
    qm = q_ref[...]
    qp = jax.lax.dot_general(qm, wq_ref[...], (((1,), (1,)), ((), ())),
                             preferred_element_type=jnp.float32) + bq_ref[...]
    scale = 1.0 / jnp.sqrt(jnp.float32(DH))
    scores = []
    vps = []
    for k in range(K):
        rk = r_ref[pl.ds(k * B * N + c * BN, BN), :]
        kp = jax.lax.dot_general(rk, wk_ref[...], (((1,), (1,)), ((), ())),
                                 preferred_element_type=jnp.float32) + bk_ref[...]
        vp = jax.lax.dot_general(rk, wv_ref[...], (((1,), (1,)), ((), ())),
                                 preferred_element_type=jnp.float32) + bv_ref[...]
        vps.append(vp)
        e = qp * kp
        scores.append(jnp.sum(e.reshape(BN, H, DH), axis=-1) * scale)
    mx = scores[0]
    for k in range(1, K):
        mx = jnp.maximum(mx, scores[k])
    exps = [jnp.exp(s - mx) for s in scores]
    z = exps[0]
    for k in range(1, K):
        z = z + exps[k]
    o = jnp.zeros((BN, D), jnp.float32)
    for k in range(K):
        a = (exps[k] / z).reshape(BN, H, 1)
        a = jnp.broadcast_to(a, (BN, H, DH)).reshape(BN, D)
        o = o + a * vps[k]
    t = jax.lax.dot_general(o, wo_ref[...], (((1,), (1,)), ((), ())),
                            preferred_element_type=jnp.float32) + bo_ref[...]
    out = jax.lax.dot_general(t, pw_ref[...], (((1,), (1,)), ((), ())),
                              preferred_element_type=jnp.float32) + pb_ref[...]
    out_ref[...] = out


def _run_attention(q, rm, wq, bq, wk, bk, wv, bv, wo, bo, proj_w, proj_b):
    full = lambda shape: pl.BlockSpec(shape, lambda c: tuple(0 for _ in shape))
    bn = (B * N) // ATT_C
    return pl.pallas_call(
        _attn_body,
        grid=(ATT_C,),
        in_specs=[
            pl.BlockSpec((bn, D), lambda c: (c, 0)),
            full((BK * N, D)),
            full((D, D)), full((1, D)), full((D, D)), full((1, D)),
            full((D, D)), full((1, D)), full((D, D)), full((1, D)),
            full((D, D)), full((1, D)),
        ],
        out_specs=pl.BlockSpec((bn, D), lambda c: (c, 0)),
        out_shape=jax.ShapeDtypeStruct((B * N, D), jnp.float32),
    )(q.reshape(B * N, D), rm, wq, bq.reshape(1, D), wk, bk.reshape(1, D),
      wv, bv.reshape(1, D), wo, bo.reshape(1, D), proj_w, proj_b.reshape(1, D))


def kernel(x_scalar, season_q, year_q, dw_w, dw_b, pw_w, pw_b, ln_g, ln_b,
           wq, bq, wk, bk, wv, bv, wo, bo, proj_w, proj_b,
           memory_bank, memory_years, memory_seasons):
    mb = memory_bank.reshape(M, ND)
    q = _run_encoder(x_scalar, dw_w, dw_b, pw_w, pw_b, ln_g, ln_b)
    idx128 = _run_sim_topk(q, mb, memory_seasons, memory_years,
                           season_q, year_q)
    idx_flat = idx128[:, :K].T.reshape(-1)
    retrieved = _run_gather(mb, idx_flat)
    rm = retrieved.reshape(BK * N, D)
    out = _run_attention(q, rm, wq, bq, wk, bk, wv, bv, wo, bo,
                         proj_w, proj_b)
    return out.reshape(B, N, D), q.reshape(B, N, D)


# speedup vs baseline: 1.3456x; 1.3456x over previous
import functools

import jax
import jax.numpy as jnp
import numpy as np
from jax.experimental import pallas as pl
from jax.experimental.pallas import tpu as pltpu

B, T, N, D, M, K, H = 16, 48, 256, 64, 4096, 8, 4
DH = D // H
ND = N * D
TAU_TIME = 2.0
TPAD = 64
CONV_K = 12
O_BLK = 2048
M_BLK = 256
BK = B * K


_ERFC_P = [2.326819970068386e-2, -1.387039388740657e-1, 3.687424674597105e-1,
           -5.824733027278666e-1, 6.210004621745983e-1, -4.944515323274145e-1,
           3.404879937665872e-1, -2.741127028184656e-1, 5.638259427386472e-1]
_ERFC_R = [-1.047766399936249e+1, 1.297719955372516e+1, -7.495518717768503e+0,
           2.921019019210786e+0, -1.015265279202700e+0, 4.218463358204948e-1,
           -2.820767439740514e-1, 5.641895067754075e-1]


def _horner(y, coeffs):
    p = jnp.full_like(y, coeffs[0])
    for c in coeffs[1:]:
        p = p * y + c
    return p


def _erfc(a):
    ax = jnp.abs(a)
    z = jnp.exp(-a * a)
    q = 1.0 / ax
    y2 = q * q
    p = jnp.where(ax < 2.0, _horner(y2, _ERFC_P), _horner(y2, _ERFC_R))
    y = z * q * p
    y = jnp.where(a * a > 88.72283905206835, 0.0, y)
    big = jnp.where(a < 0.0, 2.0 - y, y)
    small = 1.0 - jax.lax.erf(a)
    return jnp.where(ax > 1.0, big, small)


_SQRT_HALF = float(np.float32(0.7071067811865476))


def _gelu(x):
    return 0.5 * x * _erfc(-x * _SQRT_HALF)


def _encoder_body(xp_ref, wj_ref, dwb_ref, pw_ref, pwb_ref, lng_ref, lnb_ref,
                  q_ref, xg_ref):
    step = pl.program_id(0)

    @pl.when(step == 0)
    def _():
        acc = jnp.zeros((B * TPAD, N), jnp.float32)
        for j in range(CONV_K):
            acc = acc + xp_ref[j:j + B * TPAD, :] * wj_ref[j:j + 1, :]
        acc = acc + dwb_ref[...]
        xg_ref[...] = _gelu(acc)

    xg = xg_ref[...]
    y = jax.lax.dot_general(xg, pw_ref[...], (((1,), (1,)), ((), ())),
                            preferred_element_type=jnp.float32)
    y = _gelu(y + pwb_ref[...])
    row = jax.lax.broadcasted_iota(jnp.int32, (B * TPAD, 1), 0)
    valid = (row % TPAD) < (T + 1)
    y = jnp.where(valid, y, 0.0)
    qraw = jnp.sum(y.reshape(B, TPAD, O_BLK), axis=1) / jnp.float32(T + 1)
    qn = qraw.reshape(B, O_BLK // D, D)
    mu = jnp.mean(qn, axis=-1, keepdims=True)
    var = jnp.mean((qn - mu) * (qn - mu), axis=-1, keepdims=True)
    qn = (qn - mu) / jnp.sqrt(var + 1e-5)
    qn = qn * lng_ref[...].reshape(1, 1, D) + lnb_ref[...].reshape(1, 1, D)
    q_ref[...] = qn.reshape(B, O_BLK)


def _run_encoder(x_scalar, dw_w, dw_b, pw_w, pw_b, ln_g, ln_b):
    xp = jnp.pad(x_scalar, ((0, 0), (6, TPAD - 6 - T), (0, 0)))
    xp = xp.reshape(B * TPAD, N)
    xp = jnp.pad(xp, ((0, 16), (0, 0)))
    wj = jnp.pad(jnp.transpose(dw_w[:, 0, :], (1, 0)), ((0, 4), (0, 0)))
    grid = ND // O_BLK
    return pl.pallas_call(
        _encoder_body,
        grid=(grid,),
        in_specs=[
            pl.BlockSpec((B * TPAD + 16, N), lambda o: (0, 0)),
            pl.BlockSpec((16, N), lambda o: (0, 0)),
            pl.BlockSpec((1, N), lambda o: (0, 0)),
            pl.BlockSpec((O_BLK, N), lambda o: (o, 0)),
            pl.BlockSpec((1, O_BLK), lambda o: (0, o)),
            pl.BlockSpec((1, D), lambda o: (0, 0)),
            pl.BlockSpec((1, D), lambda o: (0, 0)),
        ],
        out_specs=pl.BlockSpec((B, O_BLK), lambda o: (0, o)),
        out_shape=jax.ShapeDtypeStruct((B, ND), jnp.float32),
        scratch_shapes=[pltpu.VMEM((B * TPAD, N), jnp.float32)],
    )(xp, wj, dw_b.reshape(1, N), pw_w, pw_b.reshape(1, ND),
      ln_g.reshape(1, D), ln_b.reshape(1, D))


def _sim_body(q_ref, mb_ref, msea_ref, myr_ref, seaq_ref, yrq_ref,
              idx_ref, qf_ref, sim_ref):
    step = pl.program_id(0)
    nsteps = pl.num_programs(0)

    @pl.when(step == 0)
    def _():
        q = q_ref[...]
        nrm = jnp.sqrt(jnp.sum(q * q, axis=1, keepdims=True))
        qf_ref[...] = q / jnp.maximum(nrm, 1e-12)

    mb = mb_ref[...]
    nrm = jnp.sqrt(jnp.sum(mb * mb, axis=1, keepdims=True))
    mf = mb / jnp.maximum(nrm, 1e-12)
    cos = jax.lax.dot_general(qf_ref[...], mf, (((1,), (1,)), ((), ())),
                              preferred_element_type=jnp.float32)
    smask = (seaq_ref[...] == msea_ref[...]).astype(jnp.float32)
    sim = cos * smask + (1.0 - smask) * (-10000.0)
    dy = jnp.abs(yrq_ref[...] - myr_ref[...])
    sim = sim * (1.0 - 0.5 * jnp.exp(-dy / TAU_TIME))
    sim_ref[:, pl.ds(step * M_BLK, M_BLK)] = sim

    @pl.when(step == nsteps - 1)
    def _():
        sims = sim_ref[...]
        iota = jax.lax.broadcasted_iota(jnp.int32, (B, M), 1)
        for k in range(K):
            v = jnp.max(sims, axis=1, keepdims=True)
            cand = jnp.where(sims == v, iota, M)
            sel = jnp.min(cand, axis=1, keepdims=True)
            idx_ref[:, k:k + 1] = sel
            sims = jnp.where(iota == sel, -jnp.inf, sims)


def _run_sim_topk(q, mb, memory_seasons, memory_years, season_q, year_q):
    grid = M // M_BLK
    return pl.pallas_call(
        _sim_body,
        grid=(grid,),
        in_specs=[
            pl.BlockSpec((B, ND), lambda i: (0, 0)),
            pl.BlockSpec((M_BLK, ND), lambda i: (i, 0)),
            pl.BlockSpec((1, M_BLK), lambda i: (0, i)),
            pl.BlockSpec((1, M_BLK), lambda i: (0, i)),
            pl.BlockSpec((B, 1), lambda i: (0, 0)),
            pl.BlockSpec((B, 1), lambda i: (0, 0)),
        ],
        out_specs=pl.BlockSpec((B, 128), lambda i: (0, 0)),
        out_shape=jax.ShapeDtypeStruct((B, 128), jnp.int32),
        scratch_shapes=[pltpu.VMEM((B, ND), jnp.float32),
                        pltpu.VMEM((B, M), jnp.float32)],
    )(q, mb, memory_seasons.reshape(1, M), memory_years.reshape(1, M),
      season_q.reshape(B, 1), year_q.reshape(B, 1))


def _run_gather(mb, idx_flat):
    from jax.experimental.pallas import tpu_sc as plsc

    mesh = plsc.ScalarSubcoreMesh(axis_name="core", num_cores=2)
    half = BK // 2

    @functools.partial(
        pl.kernel,
        out_type=jax.ShapeDtypeStruct((BK, ND), jnp.float32),
        mesh=mesh,
        scratch_types=[pltpu.SMEM((BK,), jnp.int32),
                       pltpu.SemaphoreType.DMA,
                       pltpu.SemaphoreType.DMA,
                       pltpu.SemaphoreType.DMA,
                       pltpu.SemaphoreType.DMA],
    )
    def gk(idx_hbm, mb_hbm, out_hbm, idx_smem, s0, s1, s2, s3):
        core = jax.lax.axis_index("core")
        sems = (s0, s1, s2, s3)
        pltpu.async_copy(idx_hbm, idx_smem, s0).wait()
        base = core * half
        copies = []
        for j in range(half):
            c = pltpu.make_async_copy(mb_hbm.at[idx_smem[base + j]],
                                      out_hbm.at[base + j], sems[j % 4])
            c.start()
            copies.append(c)
        for c in copies:
            c.wait()

    return gk(idx_flat, mb)


ATT_C = 8


def _attn_body(q_ref, r_ref, wq_ref, bq_ref, wk_ref, bk_ref, wv_ref, bv_ref,
               wo_ref, bo_ref, pw_ref, pb_ref, out_ref):
    BN = (B * N) // ATT_C
    c = pl.program_id(0)
    qm = q_ref[...]
    qp = jax.lax.dot_general(qm, wq_ref[...], (((1,), (1,)), ((), ())),
                             preferred_element_type=jnp.float32) + bq_ref[...]
    scale = 1.0 / jnp.sqrt(jnp.float32(DH))
    scores = []
    vps = []
    for k in range(K):
        rk = r_ref[pl.ds(k * B * N + c * BN, BN), :]
        kp = jax.lax.dot_general(rk, wk_ref[...], (((1,), (1,)), ((), ())),
                                 preferred_element_type=jnp.float32) + bk_ref[...]
        vp = jax.lax.dot_general(rk, wv_ref[...], (((1,), (1,)), ((), ())),
                                 preferred_element_type=jnp.float32) + bv_ref[...]
        vps.append(vp)
        e = qp * kp
        scores.append(jnp.sum(e.reshape(BN, H, DH), axis=-1) * scale)
    mx = scores[0]
    for k in range(1, K):
        mx = jnp.maximum(mx, scores[k])
    exps = [jnp.exp(s - mx) for s in scores]
    z = exps[0]
    for k in range(1, K):
        z = z + exps[k]
    o = jnp.zeros((BN, D), jnp.float32)
    for k in range(K):
        a = (exps[k] / z).reshape(BN, H, 1)
        a = jnp.broadcast_to(a, (BN, H, DH)).reshape(BN, D)
        o = o + a * vps[k]
    t = jax.lax.dot_general(o, wo_ref[...], (((1,), (1,)), ((), ())),
                            preferred_element_type=jnp.float32) + bo_ref[...]
    out = jax.lax.dot_general(t, pw_ref[...], (((1,), (1,)), ((), ())),
                              preferred_element_type=jnp.float32) + pb_ref[...]
    out_ref[...] = out


def _run_attention(q, rm, wq, bq, wk, bk, wv, bv, wo, bo, proj_w, proj_b):
    full = lambda shape: pl.BlockSpec(shape, lambda c: tuple(0 for _ in shape))
    bn = (B * N) // ATT_C
    return pl.pallas_call(
        _attn_body,
        grid=(ATT_C,),
        in_specs=[
            pl.BlockSpec((bn, D), lambda c: (c, 0)),
            full((BK * N, D)),
            full((D, D)), full((1, D)), full((D, D)), full((1, D)),
            full((D, D)), full((1, D)), full((D, D)), full((1, D)),
            full((D, D)), full((1, D)),
        ],
        out_specs=pl.BlockSpec((bn, D), lambda c: (c, 0)),
        out_shape=jax.ShapeDtypeStruct((B * N, D), jnp.float32),
    )(q.reshape(B * N, D), rm, wq, bq.reshape(1, D), wk, bk.reshape(1, D),
      wv, bv.reshape(1, D), wo, bo.reshape(1, D), proj_w, proj_b.reshape(1, D))


def kernel(x_scalar, season_q, year_q, dw_w, dw_b, pw_w, pw_b, ln_g, ln_b,
           wq, bq, wk, bk, wv, bv, wo, bo, proj_w, proj_b,
           memory_bank, memory_years, memory_seasons):
    mb = memory_bank.reshape(M, ND)
    q = _run_encoder(x_scalar, dw_w, dw_b, pw_w, pw_b, ln_g, ln_b)
    idx128 = _run_sim_topk(q, mb, memory_seasons, memory_years,
                           season_q, year_q)
    idx_flat = idx128[:, :K].T.reshape(-1)
    retrieved = _run_gather(mb, idx_flat)
    rm = retrieved.reshape(BK * N, D)
    out = _run_attention(q, rm, wq, bq, wk, bk, wv, bv, wo, bo,
                         proj_w, proj_b)
    return out.reshape(B, N, D), q.reshape(B, N, D)
